# baseline (speedup 1.0000x reference)
import jax
import jax.numpy as jnp
from jax import lax
from jax.experimental import pallas as pl
from jax.experimental.pallas import tpu as pltpu
from jax.experimental.pallas import tpu_sc as plsc

NUM_USERS = 20000
NUM_ITEMS = 30000
N = NUM_USERS + NUM_ITEMS
E = 800000
D = 64
BATCH = 4096

NC, NS, L = 2, 16, 16
NW = NC * NS

RN = 2560
NR = 20
NRT = NR * RN
SL = RN // NS
CPB = 1792
B = 64
MAXNB = (CPB + B - 1) // B

SCAN_B = 1024
NB = 26
EPW = SCAN_B * NB
E_PAD = NW * EPW

TCB = 400

_MESH = None


def _mesh():
    global _MESH
    if _MESH is None:
        _MESH = plsc.VectorSubcoreMesh(core_axis_name="c", subcore_axis_name="s")
    return _MESH


def _tc1_body(x_ref, w_ref, a_ref, outa, outb, outc):
    x = x_ref[...]
    h = jnp.dot(x, w_ref[...], preferred_element_type=jnp.float32)
    hh = h.reshape(-1, 4, 64)
    asrc = (hh * a_ref[:, :64][None]).sum(-1)
    adst = (hh * a_ref[:, 64:][None]).sum(-1)
    outa[...] = h[:, :128]
    outb[...] = h[:, 128:]
    outc[...] = jnp.concatenate(
        [asrc, adst, jnp.zeros((x.shape[0], 120), jnp.float32)], axis=1)


def _tc1(x, w1, a1):
    return pl.pallas_call(
        _tc1_body,
        grid=(N // TCB,),
        in_specs=[
            pl.BlockSpec((TCB, 64), lambda i: (i, 0)),
            pl.BlockSpec((64, 256), lambda i: (0, 0)),
            pl.BlockSpec((4, 128), lambda i: (0, 0)),
        ],
        out_specs=[
            pl.BlockSpec((TCB, 128), lambda i: (i, 0)),
            pl.BlockSpec((TCB, 128), lambda i: (i, 0)),
            pl.BlockSpec((TCB, 128), lambda i: (i, 0)),
        ],
        out_shape=[
            jax.ShapeDtypeStruct((N, 128), jnp.float32),
            jax.ShapeDtypeStruct((N, 128), jnp.float32),
            jax.ShapeDtypeStruct((N, 128), jnp.float32),
        ],
    )(x, w1, a1)


def _tc2_body(pa_ref, pb_ref, pd_ref, w_ref, a_ref, out_ref):
    pa = pa_ref[0] + pa_ref[1]
    pb = pb_ref[0] + pb_ref[1]
    pd = pd_ref[0] + pd_ref[1]
    den = pd[:, 0:4] + 1e-8
    m01 = pa.reshape(-1, 2, 64) / den[:, 0:2, None]
    m23 = pb.reshape(-1, 2, 64) / den[:, 2:4, None]
    m = jnp.concatenate([m01.reshape(-1, 128), m23.reshape(-1, 128)], axis=1)
    o = jnp.where(m > 0, m, jnp.exp(jnp.minimum(m, 0.0)) - 1.0)
    h2 = jnp.dot(o, w_ref[...], preferred_element_type=jnp.float32)
    asrc = (h2 * a_ref[0, :64][None]).sum(-1, keepdims=True)
    adst = (h2 * a_ref[0, 64:][None]).sum(-1, keepdims=True)
    out_ref[...] = jnp.concatenate(
        [h2, asrc, adst, jnp.zeros((h2.shape[0], 62), jnp.float32)], axis=1)


def _tc2(pa, pb, pd, w2, a2):
    return pl.pallas_call(
        _tc2_body,
        grid=(N // TCB,),
        in_specs=[
            pl.BlockSpec((2, TCB, 128), lambda i: (0, i, 0)),
            pl.BlockSpec((2, TCB, 128), lambda i: (0, i, 0)),
            pl.BlockSpec((2, TCB, 16), lambda i: (0, i, 0)),
            pl.BlockSpec((256, 64), lambda i: (0, 0)),
            pl.BlockSpec((1, 128), lambda i: (0, 0)),
        ],
        out_specs=pl.BlockSpec((TCB, 128), lambda i: (i, 0)),
        out_shape=jax.ShapeDtypeStruct((N, 128), jnp.float32),
    )(pa, pb, pd, w2, a2)


def _tc3_body(p_ref, x_ref, out_ref):
    p = p_ref[0] + p_ref[1]
    hf = p[:, :64] / (p[:, 64:65] + 1e-8) + x_ref[...]
    out_ref[...] = jnp.concatenate(
        [hf, jnp.zeros((hf.shape[0], 64), jnp.float32)], axis=1)


def _tc3(p2, x):
    return pl.pallas_call(
        _tc3_body,
        grid=(N // TCB,),
        in_specs=[
            pl.BlockSpec((2, TCB, 128), lambda i: (0, i, 0)),
            pl.BlockSpec((TCB, 64), lambda i: (i, 0)),
        ],
        out_specs=pl.BlockSpec((TCB, 128), lambda i: (i, 0)),
        out_shape=jax.ShapeDtypeStruct((N, 128), jnp.float32),
    )(p2, x)


def _route_body(es, ed, cpbig, counts, ebs, ebd, cp, scr, cvec):
    c = lax.axis_index("c")
    s = lax.axis_index("s")
    wid = c * NS + s
    ebase = wid * EPW
    iota = lax.iota(jnp.int32, L)
    zeros = jnp.zeros((L,), jnp.int32)
    scr[pl.ds(0, L)] = zeros
    scr[pl.ds(24, L)] = zeros

    def per_range(r, carry):
        r_lo = r * RN

        def per_block(b, cnt):
            pltpu.sync_copy(es.at[pl.ds(ebase + b * SCAN_B, SCAN_B)], ebs)
            pltpu.sync_copy(ed.at[pl.ds(ebase + b * SCAN_B, SCAN_B)], ebd)

            def per_vreg(v, cnt):
                srcv = ebs[pl.ds(v * L, L)]
                dstv = ebd[pl.ds(v * L, L)]
                loc = dstv - r_lo
                m = (loc >= 0) & (loc < RN)
                p = jnp.where(m, 1, 0)
                for sh in (1, 2, 4, 8):
                    scr[pl.ds(8, L)] = p
                    p = p + scr[pl.ds(8 - sh, L)]
                total = p[L - 1]

                def compact(cnt):
                    packed = srcv | (loc << 16)
                    d = jnp.where(m, iota - p + 1, 0)
                    vv = packed
                    for bb in (1, 2, 4, 8):
                        scr[pl.ds(8, L)] = d
                        dsh = scr[pl.ds(8 + bb, L)]
                        scr[pl.ds(8, L)] = vv
                        vsh = scr[pl.ds(8 + bb, L)]
                        mv = (dsh & bb) != 0
                        vv = jnp.where(mv, vsh, vv)
                        d = jnp.where(mv, dsh - bb, d)
                    cp[pl.ds(jnp.minimum(cnt, CPB - L), L)] = vv
                    return cnt + total

                return lax.cond(total > 0, compact, lambda q: q, cnt)

            return lax.fori_loop(0, SCAN_B // L, per_vreg, cnt)

        cnt = lax.fori_loop(0, NB, per_block, 0)
        cnt = jnp.minimum(cnt, CPB)
        pltpu.sync_copy(cp, cpbig.at[pl.ds((wid * NR + r) * CPB, CPB)])
        cvec[pl.ds(r, L)] = jnp.full((L,), cnt, jnp.int32)
        return carry

    lax.fori_loop(0, NR, per_range, 0)
    pltpu.sync_copy(cvec.at[pl.ds(0, 32)], counts.at[pl.ds(wid * 32, 32)])


def _route(es, ed):
    return pl.kernel(
        _route_body,
        out_type=(jax.ShapeDtypeStruct((NW * NR * CPB,), jnp.int32),
                  jax.ShapeDtypeStruct((NW * 32,), jnp.int32)),
        mesh=_mesh(),
        scratch_types=[
            pltpu.VMEM((SCAN_B,), jnp.int32),
            pltpu.VMEM((SCAN_B,), jnp.int32),
            pltpu.VMEM((CPB,), jnp.int32),
            pltpu.VMEM((40,), jnp.int32),
            pltpu.VMEM((NR + L,), jnp.int32),
        ],
    )(es, ed)


def _flush1_body(hxa, hxb, hxc, adt, cpbig, counts, z128, z16,
                 outa, outb, outd,
                 acca, accb, accd, atbl, ga, gb, gc, gd, cpv, sidx, didx, ldx,
                 cvbuf):
    c = lax.axis_index("c")
    s = lax.axis_index("s")
    wid = c * NS + s
    iota = lax.iota(jnp.int32, L)
    pltpu.sync_copy(counts.at[pl.ds(wid * 32, 32)], cvbuf.at[pl.ds(0, 32)])

    def per_range(r, carry):
        r_lo = r * RN
        pltpu.sync_copy(z128, acca.at[pl.ds(s * SL, SL), :])
        pltpu.sync_copy(z128, accb.at[pl.ds(s * SL, SL), :])
        pltpu.sync_copy(z16, accd.at[pl.ds(s * SL, SL), :])

        @pl.when(s == NS - 1)
        def _trash():
            pltpu.sync_copy(z128.at[pl.ds(0, L), :], acca.at[pl.ds(RN, L), :])
            pltpu.sync_copy(z128.at[pl.ds(0, L), :], accb.at[pl.ds(RN, L), :])
            pltpu.sync_copy(z16.at[pl.ds(0, L), :], accd.at[pl.ds(RN, L), :])

        pltpu.sync_copy(adt.at[pl.ds(r_lo * 4, RN * 4)], atbl.at[pl.ds(0, RN * 4)])
        pltpu.sync_copy(cpbig.at[pl.ds((wid * NR + r) * CPB, CPB)], cpv)
        plsc.subcore_barrier()
        cv = cvbuf[pl.ds(r, L)]
        cnt = cv[0]

        def per_batch(t, carry2):
            @pl.when(t * B < cnt)
            def _do():
                tb = t * B
                for k in range(B // L):
                    pv = cpv[pl.ds(tb + k * L, L)]
                    mm = (iota + (tb + k * L)) < cnt
                    sidx[pl.ds(k * L, L)] = jnp.where(mm, pv & 0xFFFF, 0)
                    lv = jnp.where(mm, pv >> 16, RN)
                    didx[pl.ds(k * L, L)] = lv
                    ldx[pl.ds(k * L, L)] = lv
                pltpu.sync_copy(hxa.at[sidx], ga)
                pltpu.sync_copy(hxb.at[sidx], gb)
                pltpu.sync_copy(hxc.at[sidx], gc)

                def edge(j, cc):
                    lvv = ldx[pl.ds(j, L)]
                    loc = lvv[0]
                    arow = gc[j, pl.ds(0, L)]
                    atv = atbl[pl.ds(4 * loc, L)]
                    lg = arow + atv
                    lr = jnp.where(lg > 0, lg, 0.2 * lg)
                    raw = jnp.exp(lr)
                    gd[j, pl.ds(0, L)] = raw
                    for h in range(4):
                        bh = raw[h]
                        tgt = ga if h < 2 else gb
                        cbase = (h % 2) * 64
                        for kk in range(4):
                            col = cbase + kk * L
                            tgt[j, pl.ds(col, L)] = tgt[j, pl.ds(col, L)] * bh
                    return cc

                lax.fori_loop(0, B, edge, 0)
                pltpu.sync_copy(ga, acca.at[didx], add=True)
                pltpu.sync_copy(gb, accb.at[didx], add=True)
                pltpu.sync_copy(gd, accd.at[didx], add=True)
            return carry2

        lax.fori_loop(0, MAXNB, per_batch, 0)
        plsc.subcore_barrier()
        rows = pl.ds(s * SL, SL)
        orows = pl.ds(r_lo + s * SL, SL)
        pltpu.sync_copy(acca.at[rows, :], outa.at[c, orows, :])
        pltpu.sync_copy(accb.at[rows, :], outb.at[c, orows, :])
        pltpu.sync_copy(accd.at[rows, :], outd.at[c, orows, :])
        plsc.subcore_barrier()
        return carry

    lax.fori_loop(0, NR, per_range, 0)


def _flush1(hxa, hxb, hxc, adt, cpbig, counts, z128, z16):
    return pl.kernel(
        _flush1_body,
        out_type=(jax.ShapeDtypeStruct((NC, NRT, 128), jnp.float32),
                  jax.ShapeDtypeStruct((NC, NRT, 128), jnp.float32),
                  jax.ShapeDtypeStruct((NC, NRT, 16), jnp.float32)),
        mesh=_mesh(),
        scratch_types=[
            pltpu.VMEM_SHARED((RN + L, 128), jnp.float32),
            pltpu.VMEM_SHARED((RN + L, 128), jnp.float32),
            pltpu.VMEM_SHARED((RN + L, 16), jnp.float32),
            pltpu.VMEM((RN * 4 + L,), jnp.float32),
            pltpu.VMEM((B, 128), jnp.float32),
            pltpu.VMEM((B, 128), jnp.float32),
            pltpu.VMEM((B, 128), jnp.float32),
            pltpu.VMEM((B, 16), jnp.float32),
            pltpu.VMEM((CPB,), jnp.int32),
            pltpu.VMEM((B,), jnp.int32),
            pltpu.VMEM((B,), jnp.int32),
            pltpu.VMEM((B + L,), jnp.int32),
            pltpu.VMEM((NR + 2 * L,), jnp.int32),
        ],
    )(hxa, hxb, hxc, adt, cpbig, counts, z128, z16)


def _flush2_body(hx2, adt, cpbig, counts, z128,
                 out2, acc, atbl, g, cpv, sidx, didx, ldx, cvbuf):
    c = lax.axis_index("c")
    s = lax.axis_index("s")
    wid = c * NS + s
    iota = lax.iota(jnp.int32, L)
    pltpu.sync_copy(counts.at[pl.ds(wid * 32, 32)], cvbuf.at[pl.ds(0, 32)])

    def per_range(r, carry):
        r_lo = r * RN
        pltpu.sync_copy(z128, acc.at[pl.ds(s * SL, SL), :])

        @pl.when(s == NS - 1)
        def _trash():
            pltpu.sync_copy(z128.at[pl.ds(0, L), :], acc.at[pl.ds(RN, L), :])

        pltpu.sync_copy(adt.at[pl.ds(r_lo * 4, RN * 4)], atbl.at[pl.ds(0, RN * 4)])
        pltpu.sync_copy(cpbig.at[pl.ds((wid * NR + r) * CPB, CPB)], cpv)
        plsc.subcore_barrier()
        cv = cvbuf[pl.ds(r, L)]
        cnt = cv[0]

        def per_batch(t, carry2):
            @pl.when(t * B < cnt)
            def _do():
                tb = t * B
                for k in range(B // L):
                    pv = cpv[pl.ds(tb + k * L, L)]
                    mm = (iota + (tb + k * L)) < cnt
                    sidx[pl.ds(k * L, L)] = jnp.where(mm, pv & 0xFFFF, 0)
                    lv = jnp.where(mm, pv >> 16, RN)
                    didx[pl.ds(k * L, L)] = lv
                    ldx[pl.ds(k * L, L)] = lv
                pltpu.sync_copy(hx2.at[sidx], g)

                def edge(j, cc):
                    lvv = ldx[pl.ds(j, L)]
                    loc = lvv[0]
                    arow = g[j, pl.ds(64, L)]
                    atv = atbl[pl.ds(4 * loc, L)]
                    lg = arow + atv
                    lr = jnp.where(lg > 0, lg, 0.2 * lg)
                    raw = jnp.exp(lr)
                    g[j, pl.ds(64, L)] = raw
                    bh = raw[0]
                    for kk in range(4):
                        col = kk * L
                        g[j, pl.ds(col, L)] = g[j, pl.ds(col, L)] * bh
                    return cc

                lax.fori_loop(0, B, edge, 0)
                pltpu.sync_copy(g, acc.at[didx], add=True)
            return carry2

        lax.fori_loop(0, MAXNB, per_batch, 0)
        plsc.subcore_barrier()
        pltpu.sync_copy(acc.at[pl.ds(s * SL, SL), :],
                        out2.at[c, pl.ds(r_lo + s * SL, SL), :])
        plsc.subcore_barrier()
        return carry

    lax.fori_loop(0, NR, per_range, 0)


def _flush2(hx2, adt, cpbig, counts, z128):
    return pl.kernel(
        _flush2_body,
        out_type=jax.ShapeDtypeStruct((NC, NRT, 128), jnp.float32),
        mesh=_mesh(),
        scratch_types=[
            pltpu.VMEM_SHARED((RN + L, 128), jnp.float32),
            pltpu.VMEM((RN * 4 + L,), jnp.float32),
            pltpu.VMEM((B, 128), jnp.float32),
            pltpu.VMEM((CPB,), jnp.int32),
            pltpu.VMEM((B,), jnp.int32),
            pltpu.VMEM((B,), jnp.int32),
            pltpu.VMEM((B + L,), jnp.int32),
            pltpu.VMEM((NR + 2 * L,), jnp.int32),
        ],
    )(hx2, adt, cpbig, counts, z128)


PB = BATCH // NW
CH = 32


def _dot_body(hf, uids, iids, out, ub, ib, ur, ir, ob, fscr):
    c = lax.axis_index("c")
    s = lax.axis_index("s")
    base = (c * NS + s) * PB
    zf = jnp.zeros((L,), jnp.float32)
    fscr[pl.ds(0, L)] = zf
    fscr[pl.ds(24, L)] = zf

    def chunk(q, carry):
        cb = base + q * CH
        pltpu.sync_copy(uids.at[pl.ds(cb, CH)], ub)
        pltpu.sync_copy(iids.at[pl.ds(cb, CH)], ib)
        pltpu.sync_copy(hf.at[ub], ur)
        pltpu.sync_copy(hf.at[ib], ir)

        def pair(j, carry2):
            acc = ur[j, pl.ds(0, L)] * ir[j, pl.ds(0, L)]
            for k in range(1, D // L):
                acc = acc + ur[j, pl.ds(k * L, L)] * ir[j, pl.ds(k * L, L)]
            for sh in (1, 2, 4, 8):
                fscr[pl.ds(8, L)] = acc
                acc = acc + fscr[pl.ds(8 - sh, L)]
            total = acc[L - 1]
            ob[pl.ds(j, L)] = jnp.full((L,), total, jnp.float32)
            return carry2

        lax.fori_loop(0, CH, pair, 0)
        pltpu.sync_copy(ob.at[pl.ds(0, CH)], out.at[pl.ds(cb, CH)])
        return carry

    lax.fori_loop(0, PB // CH, chunk, 0)


def _dot(hf, uids, iids):
    return pl.kernel(
        _dot_body,
        out_type=jax.ShapeDtypeStruct((BATCH,), jnp.float32),
        mesh=_mesh(),
        scratch_types=[
            pltpu.VMEM((CH,), jnp.int32),
            pltpu.VMEM((CH,), jnp.int32),
            pltpu.VMEM((CH, 128), jnp.float32),
            pltpu.VMEM((CH, 128), jnp.float32),
            pltpu.VMEM((CH + L,), jnp.float32),
            pltpu.VMEM((40,), jnp.float32),
        ],
    )(hf, uids, iids)


@jax.jit
def kernel(user_emb, item_emb, W1, a1, W2, a2, edge_index, user_ids, item_ids):
    x = jnp.concatenate([user_emb, item_emb], axis=0)
    es = jnp.pad(edge_index[0], (0, E_PAD - E))
    ed = jnp.pad(edge_index[1], (0, E_PAD - E), constant_values=2 ** 24)
    z128 = jnp.zeros((SL, 128), jnp.float32)
    z16 = jnp.zeros((SL, 16), jnp.float32)

    cpbig, counts = _route(es, ed)

    hxa, hxb, hxc = _tc1(x, W1, a1)
    adt1 = jnp.pad(hxc[:, 4:8], ((0, NRT - N), (0, 0))).reshape(-1)
    pa, pb, pd = _flush1(hxa, hxb, hxc, adt1, cpbig, counts, z128, z16)

    hx2 = _tc2(pa, pb, pd, W2, a2)
    adt2 = jnp.pad(hx2[:, 65:66], ((0, NRT - N), (0, 3))).reshape(-1)
    p2 = _flush2(hx2, adt2, cpbig, counts, z128)

    hf = _tc3(p2, x)
    return _dot(hf, user_ids, item_ids + NUM_USERS)


# speedup vs baseline: 1.0433x; 1.0433x over previous
import jax
import jax.numpy as jnp
from jax import lax
from jax.experimental import pallas as pl
from jax.experimental.pallas import tpu as pltpu
from jax.experimental.pallas import tpu_sc as plsc

NUM_USERS = 20000
NUM_ITEMS = 30000
N = NUM_USERS + NUM_ITEMS
E = 800000
D = 64
BATCH = 4096

NC, NS, L = 2, 16, 16
NW = NC * NS

RN = 2560
NR = 20
NRT = NR * RN
SL = RN // NS
CPB = 1792
B = 64
MAXNB = (CPB + B - 1) // B

SCAN_B = 1024
NB = 26
EPW = SCAN_B * NB
E_PAD = NW * EPW

TCB = 400

_MESH = None


def _mesh():
    global _MESH
    if _MESH is None:
        _MESH = plsc.VectorSubcoreMesh(core_axis_name="c", subcore_axis_name="s")
    return _MESH


def _tc1_body(x_ref, w_ref, a_ref, outa, outb, outc):
    x = x_ref[...]
    h = jnp.dot(x, w_ref[...], preferred_element_type=jnp.float32)
    hh = h.reshape(-1, 4, 64)
    asrc = (hh * a_ref[:, :64][None]).sum(-1)
    adst = (hh * a_ref[:, 64:][None]).sum(-1)
    outa[...] = h[:, :128]
    outb[...] = h[:, 128:]
    outc[...] = jnp.concatenate(
        [asrc, adst, jnp.zeros((x.shape[0], 120), jnp.float32)], axis=1)


def _tc1(x, w1, a1):
    return pl.pallas_call(
        _tc1_body,
        grid=(N // TCB,),
        in_specs=[
            pl.BlockSpec((TCB, 64), lambda i: (i, 0)),
            pl.BlockSpec((64, 256), lambda i: (0, 0)),
            pl.BlockSpec((4, 128), lambda i: (0, 0)),
        ],
        out_specs=[
            pl.BlockSpec((TCB, 128), lambda i: (i, 0)),
            pl.BlockSpec((TCB, 128), lambda i: (i, 0)),
            pl.BlockSpec((TCB, 128), lambda i: (i, 0)),
        ],
        out_shape=[
            jax.ShapeDtypeStruct((N, 128), jnp.float32),
            jax.ShapeDtypeStruct((N, 128), jnp.float32),
            jax.ShapeDtypeStruct((N, 128), jnp.float32),
        ],
    )(x, w1, a1)


def _tc2_body(pa_ref, pb_ref, pd_ref, w_ref, a_ref, out_ref):
    pa = pa_ref[0] + pa_ref[1]
    pb = pb_ref[0] + pb_ref[1]
    pd = pd_ref[0] + pd_ref[1]
    den = pd[:, 0:4] + 1e-8
    m01 = pa.reshape(-1, 2, 64) / den[:, 0:2, None]
    m23 = pb.reshape(-1, 2, 64) / den[:, 2:4, None]
    m = jnp.concatenate([m01.reshape(-1, 128), m23.reshape(-1, 128)], axis=1)
    o = jnp.where(m > 0, m, jnp.exp(jnp.minimum(m, 0.0)) - 1.0)
    h2 = jnp.dot(o, w_ref[...], preferred_element_type=jnp.float32)
    asrc = (h2 * a_ref[0, :64][None]).sum(-1, keepdims=True)
    adst = (h2 * a_ref[0, 64:][None]).sum(-1, keepdims=True)
    out_ref[...] = jnp.concatenate(
        [h2, asrc, adst, jnp.zeros((h2.shape[0], 62), jnp.float32)], axis=1)


def _tc2(pa, pb, pd, w2, a2):
    return pl.pallas_call(
        _tc2_body,
        grid=(N // TCB,),
        in_specs=[
            pl.BlockSpec((2, TCB, 128), lambda i: (0, i, 0)),
            pl.BlockSpec((2, TCB, 128), lambda i: (0, i, 0)),
            pl.BlockSpec((2, TCB, 16), lambda i: (0, i, 0)),
            pl.BlockSpec((256, 64), lambda i: (0, 0)),
            pl.BlockSpec((1, 128), lambda i: (0, 0)),
        ],
        out_specs=pl.BlockSpec((TCB, 128), lambda i: (i, 0)),
        out_shape=jax.ShapeDtypeStruct((N, 128), jnp.float32),
    )(pa, pb, pd, w2, a2)


def _tc3_body(p_ref, x_ref, out_ref):
    p = p_ref[0] + p_ref[1]
    hf = p[:, :64] / (p[:, 64:65] + 1e-8) + x_ref[...]
    out_ref[...] = jnp.concatenate(
        [hf, jnp.zeros((hf.shape[0], 64), jnp.float32)], axis=1)


def _tc3(p2, x):
    return pl.pallas_call(
        _tc3_body,
        grid=(N // TCB,),
        in_specs=[
            pl.BlockSpec((2, TCB, 128), lambda i: (0, i, 0)),
            pl.BlockSpec((TCB, 64), lambda i: (i, 0)),
        ],
        out_specs=pl.BlockSpec((TCB, 128), lambda i: (i, 0)),
        out_shape=jax.ShapeDtypeStruct((N, 128), jnp.float32),
    )(p2, x)


def _route_body(es, ed, cpbig, counts, ebs, ebd, cp, scr, cvec):
    c = lax.axis_index("c")
    s = lax.axis_index("s")
    wid = c * NS + s
    ebase = wid * EPW
    iota = lax.iota(jnp.int32, L)
    zeros = jnp.zeros((L,), jnp.int32)
    scr[pl.ds(0, L)] = zeros
    scr[pl.ds(24, L)] = zeros

    def per_range(r, carry):
        r_lo = r * RN

        def per_block(b, cnt):
            pltpu.sync_copy(es.at[pl.ds(ebase + b * SCAN_B, SCAN_B)], ebs)
            pltpu.sync_copy(ed.at[pl.ds(ebase + b * SCAN_B, SCAN_B)], ebd)

            def per_vreg(v, cnt):
                srcv = ebs[pl.ds(v * L, L)]
                dstv = ebd[pl.ds(v * L, L)]
                loc = dstv - r_lo
                m = (loc >= 0) & (loc < RN)
                p = jnp.where(m, 1, 0)
                for sh in (1, 2, 4, 8):
                    scr[pl.ds(8, L)] = p
                    p = p + scr[pl.ds(8 - sh, L)]
                total = p[L - 1]
                packed = srcv | (loc << 16)
                d = jnp.where(m, iota - p + 1, 0)
                vv = packed
                for bb in (1, 2, 4, 8):
                    scr[pl.ds(8, L)] = d
                    dsh = scr[pl.ds(8 + bb, L)]
                    scr[pl.ds(8, L)] = vv
                    vsh = scr[pl.ds(8 + bb, L)]
                    mv = (dsh & bb) != 0
                    vv = jnp.where(mv, vsh, vv)
                    d = jnp.where(mv, dsh - bb, d)
                cp[pl.ds(jnp.minimum(cnt, CPB - L), L)] = vv
                return cnt + total

            return lax.fori_loop(0, SCAN_B // L, per_vreg, cnt)

        cnt = lax.fori_loop(0, NB, per_block, 0)
        cnt = jnp.minimum(cnt, CPB)
        pltpu.sync_copy(cp, cpbig.at[pl.ds((wid * NR + r) * CPB, CPB)])
        cvec[pl.ds(r, L)] = jnp.full((L,), cnt, jnp.int32)
        return carry

    lax.fori_loop(0, NR, per_range, 0)
    pltpu.sync_copy(cvec.at[pl.ds(0, 32)], counts.at[pl.ds(wid * 32, 32)])


def _route(es, ed):
    return pl.kernel(
        _route_body,
        out_type=(jax.ShapeDtypeStruct((NW * NR * CPB,), jnp.int32),
                  jax.ShapeDtypeStruct((NW * 32,), jnp.int32)),
        mesh=_mesh(),
        scratch_types=[
            pltpu.VMEM((SCAN_B,), jnp.int32),
            pltpu.VMEM((SCAN_B,), jnp.int32),
            pltpu.VMEM((CPB,), jnp.int32),
            pltpu.VMEM((40,), jnp.int32),
            pltpu.VMEM((NR + L,), jnp.int32),
        ],
    )(es, ed)


def _flush1_body(hxa, hxb, hxc, adt, cpbig, counts, z128, z16,
                 outa, outb, outd,
                 acca, accb, accd, atbl, ga, gb, gc, gd, cpv, sidx, didx, ldx,
                 cvbuf):
    c = lax.axis_index("c")
    s = lax.axis_index("s")
    wid = c * NS + s
    iota = lax.iota(jnp.int32, L)
    pltpu.sync_copy(counts.at[pl.ds(wid * 32, 32)], cvbuf.at[pl.ds(0, 32)])

    def per_range(r, carry):
        r_lo = r * RN
        pltpu.sync_copy(z128, acca.at[pl.ds(s * SL, SL), :])
        pltpu.sync_copy(z128, accb.at[pl.ds(s * SL, SL), :])
        pltpu.sync_copy(z16, accd.at[pl.ds(s * SL, SL), :])

        @pl.when(s == NS - 1)
        def _trash():
            pltpu.sync_copy(z128.at[pl.ds(0, L), :], acca.at[pl.ds(RN, L), :])
            pltpu.sync_copy(z128.at[pl.ds(0, L), :], accb.at[pl.ds(RN, L), :])
            pltpu.sync_copy(z16.at[pl.ds(0, L), :], accd.at[pl.ds(RN, L), :])

        pltpu.sync_copy(adt.at[pl.ds(r_lo * 4, RN * 4)], atbl.at[pl.ds(0, RN * 4)])
        pltpu.sync_copy(cpbig.at[pl.ds((wid * NR + r) * CPB, CPB)], cpv)
        plsc.subcore_barrier()
        cv = cvbuf[pl.ds(r, L)]
        cnt = cv[0]

        def per_batch(t, carry2):
            @pl.when(t * B < cnt)
            def _do():
                tb = t * B
                for k in range(B // L):
                    pv = cpv[pl.ds(tb + k * L, L)]
                    mm = (iota + (tb + k * L)) < cnt
                    sidx[pl.ds(k * L, L)] = jnp.where(mm, pv & 0xFFFF, 0)
                    lv = jnp.where(mm, pv >> 16, RN)
                    didx[pl.ds(k * L, L)] = lv
                    ldx[pl.ds(k * L, L)] = lv
                pltpu.sync_copy(hxa.at[sidx], ga)
                pltpu.sync_copy(hxb.at[sidx], gb)
                pltpu.sync_copy(hxc.at[sidx], gc)

                def edge(j, cc):
                    lvv = ldx[pl.ds(j, L)]
                    loc = lvv[0]
                    arow = gc[j, pl.ds(0, L)]
                    atv = atbl[pl.ds(4 * loc, L)]
                    lg = arow + atv
                    lr = jnp.where(lg > 0, lg, 0.2 * lg)
                    raw = jnp.exp(lr)
                    gd[j, pl.ds(0, L)] = raw
                    for h in range(4):
                        bh = raw[h]
                        tgt = ga if h < 2 else gb
                        cbase = (h % 2) * 64
                        for kk in range(4):
                            col = cbase + kk * L
                            tgt[j, pl.ds(col, L)] = tgt[j, pl.ds(col, L)] * bh
                    return cc

                lax.fori_loop(0, B, edge, 0)
                pltpu.sync_copy(ga, acca.at[didx], add=True)
                pltpu.sync_copy(gb, accb.at[didx], add=True)
                pltpu.sync_copy(gd, accd.at[didx], add=True)
            return carry2

        lax.fori_loop(0, MAXNB, per_batch, 0)
        plsc.subcore_barrier()
        rows = pl.ds(s * SL, SL)
        orows = pl.ds(r_lo + s * SL, SL)
        pltpu.sync_copy(acca.at[rows, :], outa.at[c, orows, :])
        pltpu.sync_copy(accb.at[rows, :], outb.at[c, orows, :])
        pltpu.sync_copy(accd.at[rows, :], outd.at[c, orows, :])
        plsc.subcore_barrier()
        return carry

    lax.fori_loop(0, NR, per_range, 0)


def _flush1(hxa, hxb, hxc, adt, cpbig, counts, z128, z16):
    return pl.kernel(
        _flush1_body,
        out_type=(jax.ShapeDtypeStruct((NC, NRT, 128), jnp.float32),
                  jax.ShapeDtypeStruct((NC, NRT, 128), jnp.float32),
                  jax.ShapeDtypeStruct((NC, NRT, 16), jnp.float32)),
        mesh=_mesh(),
        scratch_types=[
            pltpu.VMEM_SHARED((RN + L, 128), jnp.float32),
            pltpu.VMEM_SHARED((RN + L, 128), jnp.float32),
            pltpu.VMEM_SHARED((RN + L, 16), jnp.float32),
            pltpu.VMEM((RN * 4 + L,), jnp.float32),
            pltpu.VMEM((B, 128), jnp.float32),
            pltpu.VMEM((B, 128), jnp.float32),
            pltpu.VMEM((B, 128), jnp.float32),
            pltpu.VMEM((B, 16), jnp.float32),
            pltpu.VMEM((CPB,), jnp.int32),
            pltpu.VMEM((B,), jnp.int32),
            pltpu.VMEM((B,), jnp.int32),
            pltpu.VMEM((B + L,), jnp.int32),
            pltpu.VMEM((NR + 2 * L,), jnp.int32),
        ],
    )(hxa, hxb, hxc, adt, cpbig, counts, z128, z16)


def _flush2_body(hx2, adt, cpbig, counts, z128,
                 out2, acc, atbl, g, cpv, sidx, didx, ldx, cvbuf):
    c = lax.axis_index("c")
    s = lax.axis_index("s")
    wid = c * NS + s
    iota = lax.iota(jnp.int32, L)
    pltpu.sync_copy(counts.at[pl.ds(wid * 32, 32)], cvbuf.at[pl.ds(0, 32)])

    def per_range(r, carry):
        r_lo = r * RN
        pltpu.sync_copy(z128, acc.at[pl.ds(s * SL, SL), :])

        @pl.when(s == NS - 1)
        def _trash():
            pltpu.sync_copy(z128.at[pl.ds(0, L), :], acc.at[pl.ds(RN, L), :])

        pltpu.sync_copy(adt.at[pl.ds(r_lo * 4, RN * 4)], atbl.at[pl.ds(0, RN * 4)])
        pltpu.sync_copy(cpbig.at[pl.ds((wid * NR + r) * CPB, CPB)], cpv)
        plsc.subcore_barrier()
        cv = cvbuf[pl.ds(r, L)]
        cnt = cv[0]

        def per_batch(t, carry2):
            @pl.when(t * B < cnt)
            def _do():
                tb = t * B
                for k in range(B // L):
                    pv = cpv[pl.ds(tb + k * L, L)]
                    mm = (iota + (tb + k * L)) < cnt
                    sidx[pl.ds(k * L, L)] = jnp.where(mm, pv & 0xFFFF, 0)
                    lv = jnp.where(mm, pv >> 16, RN)
                    didx[pl.ds(k * L, L)] = lv
                    ldx[pl.ds(k * L, L)] = lv
                pltpu.sync_copy(hx2.at[sidx], g)

                def edge(j, cc):
                    lvv = ldx[pl.ds(j, L)]
                    loc = lvv[0]
                    arow = g[j, pl.ds(64, L)]
                    atv = atbl[pl.ds(4 * loc, L)]
                    lg = arow + atv
                    lr = jnp.where(lg > 0, lg, 0.2 * lg)
                    raw = jnp.exp(lr)
                    g[j, pl.ds(64, L)] = raw
                    bh = raw[0]
                    for kk in range(4):
                        col = kk * L
                        g[j, pl.ds(col, L)] = g[j, pl.ds(col, L)] * bh
                    return cc

                lax.fori_loop(0, B, edge, 0)
                pltpu.sync_copy(g, acc.at[didx], add=True)
            return carry2

        lax.fori_loop(0, MAXNB, per_batch, 0)
        plsc.subcore_barrier()
        pltpu.sync_copy(acc.at[pl.ds(s * SL, SL), :],
                        out2.at[c, pl.ds(r_lo + s * SL, SL), :])
        plsc.subcore_barrier()
        return carry

    lax.fori_loop(0, NR, per_range, 0)


def _flush2(hx2, adt, cpbig, counts, z128):
    return pl.kernel(
        _flush2_body,
        out_type=jax.ShapeDtypeStruct((NC, NRT, 128), jnp.float32),
        mesh=_mesh(),
        scratch_types=[
            pltpu.VMEM_SHARED((RN + L, 128), jnp.float32),
            pltpu.VMEM((RN * 4 + L,), jnp.float32),
            pltpu.VMEM((B, 128), jnp.float32),
            pltpu.VMEM((CPB,), jnp.int32),
            pltpu.VMEM((B,), jnp.int32),
            pltpu.VMEM((B,), jnp.int32),
            pltpu.VMEM((B + L,), jnp.int32),
            pltpu.VMEM((NR + 2 * L,), jnp.int32),
        ],
    )(hx2, adt, cpbig, counts, z128)


PB = BATCH // NW
CH = 32


def _dot_body(hf, uids, iids, out, ub, ib, ur, ir, ob, fscr):
    c = lax.axis_index("c")
    s = lax.axis_index("s")
    base = (c * NS + s) * PB
    zf = jnp.zeros((L,), jnp.float32)
    fscr[pl.ds(0, L)] = zf
    fscr[pl.ds(24, L)] = zf

    def chunk(q, carry):
        cb = base + q * CH
        pltpu.sync_copy(uids.at[pl.ds(cb, CH)], ub)
        pltpu.sync_copy(iids.at[pl.ds(cb, CH)], ib)
        pltpu.sync_copy(hf.at[ub], ur)
        pltpu.sync_copy(hf.at[ib], ir)

        def pair(j, carry2):
            acc = ur[j, pl.ds(0, L)] * ir[j, pl.ds(0, L)]
            for k in range(1, D // L):
                acc = acc + ur[j, pl.ds(k * L, L)] * ir[j, pl.ds(k * L, L)]
            for sh in (1, 2, 4, 8):
                fscr[pl.ds(8, L)] = acc
                acc = acc + fscr[pl.ds(8 - sh, L)]
            total = acc[L - 1]
            ob[pl.ds(j, L)] = jnp.full((L,), total, jnp.float32)
            return carry2

        lax.fori_loop(0, CH, pair, 0)
        pltpu.sync_copy(ob.at[pl.ds(0, CH)], out.at[pl.ds(cb, CH)])
        return carry

    lax.fori_loop(0, PB // CH, chunk, 0)


def _dot(hf, uids, iids):
    return pl.kernel(
        _dot_body,
        out_type=jax.ShapeDtypeStruct((BATCH,), jnp.float32),
        mesh=_mesh(),
        scratch_types=[
            pltpu.VMEM((CH,), jnp.int32),
            pltpu.VMEM((CH,), jnp.int32),
            pltpu.VMEM((CH, 128), jnp.float32),
            pltpu.VMEM((CH, 128), jnp.float32),
            pltpu.VMEM((CH + L,), jnp.float32),
            pltpu.VMEM((40,), jnp.float32),
        ],
    )(hf, uids, iids)


@jax.jit
def kernel(user_emb, item_emb, W1, a1, W2, a2, edge_index, user_ids, item_ids):
    x = jnp.concatenate([user_emb, item_emb], axis=0)
    es = jnp.pad(edge_index[0], (0, E_PAD - E))
    ed = jnp.pad(edge_index[1], (0, E_PAD - E), constant_values=2 ** 24)
    z128 = jnp.zeros((SL, 128), jnp.float32)
    z16 = jnp.zeros((SL, 16), jnp.float32)

    cpbig, counts = _route(es, ed)

    hxa, hxb, hxc = _tc1(x, W1, a1)
    adt1 = jnp.pad(hxc[:, 4:8], ((0, NRT - N), (0, 0))).reshape(-1)
    pa, pb, pd = _flush1(hxa, hxb, hxc, adt1, cpbig, counts, z128, z16)

    hx2 = _tc2(pa, pb, pd, W2, a2)
    adt2 = jnp.pad(hx2[:, 65:66], ((0, NRT - N), (0, 3))).reshape(-1)
    p2 = _flush2(hx2, adt2, cpbig, counts, z128)

    hf = _tc3(p2, x)
    return _dot(hf, user_ids, item_ids + NUM_USERS)
